# Initial kernel scaffold; baseline (speedup 1.0000x reference)
import jax
import jax.numpy as jnp
from jax.experimental import pallas as pl


def kernel(input_matrix, adjacency_coo_matrix, W):
    raise NotImplementedError("write your pallas kernel here")


# speedup vs baseline: 4.4836x; 4.4836x over previous
import jax
import jax.numpy as jnp
from jax import lax
from jax.experimental import pallas as pl
from jax.experimental.pallas import tpu as pltpu
from jax.experimental.pallas import tpu_sc as plsc

N_NODES = 10000
D = 128
NC, NS = 2, 16
H = D // NC
NPAD = 10240
RPT = NPAD // NS
CH = 128
TRASH = N_NODES + 8


def _sc_aggregate(x2, src2, dst2):
  nrows = src2.shape[0]
  cpt = nrows // NS
  mesh = plsc.VectorSubcoreMesh(
      core_axis_name="c", subcore_axis_name="s",
      num_cores=NC, num_subcores=NS)

  def body(x_hbm, src_hbm, dst_hbm, agg_hbm, deg_hbm,
           acc, degacc, src_v, dst_v, rows_v, ones_v, zrow_v, zdeg_v):
    c = lax.axis_index("c")
    s = lax.axis_index("s")

    z16 = jnp.zeros((16,), jnp.float32)
    o16 = jnp.ones((16,), jnp.float32)

    def zrow_body(i, carry):
      for u in range(H // 16):
        zrow_v[i, pl.ds(u * 16, 16)] = z16
      return carry
    lax.fori_loop(0, zrow_v.shape[0], zrow_body, 0)

    def zdeg_body(i, carry):
      zdeg_v[pl.ds(i * 16, 16)] = z16
      return carry
    lax.fori_loop(0, RPT // 16, zdeg_body, 0)

    def ones_body(i, carry):
      ones_v[pl.ds(i * 16, 16)] = o16
      return carry
    lax.fori_loop(0, CH // 16, ones_body, 0)

    base_r = s * RPT
    half = zrow_v.shape[0]
    pltpu.sync_copy(zrow_v, acc.at[pl.ds(base_r, half)])
    pltpu.sync_copy(zrow_v, acc.at[pl.ds(base_r + half, half)])
    pltpu.sync_copy(zdeg_v, degacc.at[pl.ds(base_r, RPT)])

    row0 = s * cpt
    pltpu.sync_copy(src_hbm.at[pl.ds(row0, cpt)], src_v)
    pltpu.sync_copy(dst_hbm.at[pl.ds(row0, cpt)], dst_v)

    plsc.subcore_barrier()

    xc = x_hbm.at[c]

    def edge_body(k, carry):
      pltpu.sync_copy(xc.at[dst_v.at[k]], rows_v)
      pltpu.sync_copy(rows_v, acc.at[src_v.at[k]], add=True)
      pltpu.sync_copy(ones_v, degacc.at[src_v.at[k]], add=True)
      return carry
    lax.fori_loop(0, cpt, edge_body, 0)

    plsc.subcore_barrier()

    pltpu.sync_copy(acc.at[pl.ds(base_r, RPT)],
                    agg_hbm.at[c, pl.ds(base_r, RPT)])
    pltpu.sync_copy(degacc.at[pl.ds(base_r, RPT)],
                    deg_hbm.at[c, pl.ds(base_r, RPT)])

  run = pl.kernel(
      body,
      out_type=[
          jax.ShapeDtypeStruct((NC, NPAD, H), jnp.float32),
          jax.ShapeDtypeStruct((NC, NPAD), jnp.float32),
      ],
      mesh=mesh,
      scratch_types=[
          pltpu.VMEM_SHARED((NPAD, H), jnp.float32),
          pltpu.VMEM_SHARED((NPAD,), jnp.float32),
          pltpu.VMEM((cpt, CH), jnp.int32),
          pltpu.VMEM((cpt, CH), jnp.int32),
          pltpu.VMEM((CH, H), jnp.float32),
          pltpu.VMEM((CH,), jnp.float32),
          pltpu.VMEM((RPT // 2, H), jnp.float32),
          pltpu.VMEM((RPT,), jnp.float32),
      ],
      compiler_params=pltpu.CompilerParams(use_tc_tiling_on_sc=False),
  )
  return run(x2, src2, dst2)


def _tc_finish(agg, deg, W):
  blk = 640

  def body(a_ref, d_ref, w_ref, o_ref):
    a = jnp.concatenate([a_ref[0], a_ref[1]], axis=1)
    d = d_ref[0]
    inv = 1.0 / jnp.maximum(d, 1.0)
    m = a * inv[:, None]
    o_ref[...] = jnp.maximum(
        jnp.dot(m, w_ref[...], preferred_element_type=jnp.float32), 0.0)

  return pl.pallas_call(
      body,
      grid=(NPAD // blk,),
      in_specs=[
          pl.BlockSpec((NC, blk, H), lambda i: (0, i, 0)),
          pl.BlockSpec((NC, blk), lambda i: (0, i)),
          pl.BlockSpec((D, D), lambda i: (0, 0)),
      ],
      out_specs=pl.BlockSpec((blk, D), lambda i: (i, 0)),
      out_shape=jax.ShapeDtypeStruct((NPAD, D), jnp.float32),
  )(agg, deg, W)


def kernel(input_matrix, adjacency_coo_matrix, W):
  n_edges = adjacency_coo_matrix.shape[1]
  quantum = NS * CH * 8
  epad = ((n_edges + quantum - 1) // quantum) * quantum
  pad = epad - n_edges
  src = adjacency_coo_matrix[0]
  dst = adjacency_coo_matrix[1]
  src_p = jnp.concatenate(
      [src, jnp.full((pad,), TRASH, jnp.int32)]).reshape(epad // CH, CH)
  dst_p = jnp.concatenate(
      [dst, jnp.zeros((pad,), jnp.int32)]).reshape(epad // CH, CH)
  x2 = jnp.transpose(input_matrix.reshape(N_NODES, NC, H), (1, 0, 2))
  agg, deg = _sc_aggregate(x2, src_p, dst_p)
  out = _tc_finish(agg, deg, W)
  return out[:N_NODES]
